# Initial kernel scaffold; baseline (speedup 1.0000x reference)
import jax
import jax.numpy as jnp
from jax.experimental import pallas as pl


def kernel(x_customer, x_product, x_store, Wc, bc, Wp, bp, Ws, bs, Wl, bl, Wr, ln_g, ln_b, edge_index_buys, edge_index_bought_by, edge_index_visits, edge_index_visited_by, edge_index_sold_at, edge_index_sells):
    raise NotImplementedError("write your pallas kernel here")


# speedup vs baseline: 4.0311x; 4.0311x over previous
import functools
import math

import jax
import jax.numpy as jnp
from jax import lax
from jax.experimental import pallas as pl
from jax.experimental.pallas import tpu as pltpu
from jax.experimental.pallas import tpu_sc as plsc

H = 128
NC, NP, NS = 100000, 50000, 1000
N_T = [NC, NP, NS]
OFF = [0, NC, NC + NP]
NT_ALL = NC + NP + NS

SRC_T = [0, 1, 0, 2, 1, 2]
DST_T = [1, 0, 2, 0, 2, 1]
E_CNT = [600000, 600000, 400000, 400000, 100000, 100000]
DES = {0: (1, 3), 1: (0, 5), 2: (2, 4)}

NPAD_T = [100352, 50176, 1024]

SCRATCH_ROWS = 103552
DUMMY_ROW = 103424
ZROWS = 512
ZPS = SCRATCH_ROWS // 16

PASSES = [
    dict(members=[(1, 0, 100352), (2, 100352, 1024), (4, 101376, 1024)],
         ep=1114112),
    dict(members=[(3, 0, 100352)], ep=409600),
    dict(members=[(0, 0, 50176), (5, 50176, 50176)], ep=704512),
]
GS = [p["ep"] // 16384 for p in PASSES]
EOUT = [1, 2, 4, 3, 0, 5]

_MESH = plsc.VectorSubcoreMesh(core_axis_name="c", subcore_axis_name="s")
_SC_PARAMS = pltpu.CompilerParams(use_tc_tiling_on_sc=False)


def _zero_own(acc, zbuf, sid):
    zb = sid * ZPS
    for k in range(12):
        pltpu.sync_copy(zbuf, acc.at[pl.ds(zb + k * ZROWS, ZROWS)])
    pltpu.sync_copy(zbuf.at[pl.ds(0, ZPS - 12 * ZROWS)],
                    acc.at[pl.ds(zb + 12 * ZROWS, ZPS - 12 * ZROWS)])


def _init_zbuf(zbuf):
    def zinit(i, carry):
        zbuf[i, :] = jnp.zeros((16,), jnp.float32)
        return carry
    lax.fori_loop(0, ZROWS, zinit, 0)


def _sc_l0_body(x16, s0, d0, s1, d1, s2, d2,
                o1, o2, o4, o3, o0, o5, acc, isrc, idst, rows, zbuf, sem):
    c = lax.axis_index("c")
    sid = lax.axis_index("s")
    w = c * 16 + sid
    _init_zbuf(zbuf)
    srcs, dsts = [s0, s1, s2], [d0, d1, d2]
    outs = {1: o1, 2: o2, 4: o4, 3: o3, 0: o0, 5: o5}
    for p in range(3):
        _zero_own(acc, zbuf, sid)
        plsc.subcore_barrier()
        G = GS[p]
        rpw = G * 4
        sp, dp = srcs[p], dsts[p]

        def grp(g, carry):
            gb = w * rpw + g * 4
            pltpu.sync_copy(sp.at[pl.ds(gb, 4)], isrc)
            pltpu.sync_copy(dp.at[pl.ds(gb, 4)], idst)
            hs = [pltpu.async_copy(x16.at[isrc.at[j]], rows.at[j], sem)
                  for j in range(4)]
            for h in hs:
                h.wait()
            for j in range(4):
                pltpu.sync_copy(rows.at[j], acc.at[idst.at[j]], add=True)
            return carry

        lax.fori_loop(0, G, grp, 0)
        plsc.subcore_barrier()
        for (e, loff, npd) in PASSES[p]["members"]:
            sh = npd // 16
            pltpu.sync_copy(acc.at[pl.ds(loff + sid * sh, sh)],
                            outs[e].at[c, pl.ds(sid * sh, sh), :])
        plsc.subcore_barrier()


def _sc_l1_body(tab8, s0, d0, s1, d1, s2, d2,
                o1, o2, o4, o3, o0, o5, acc, isrc, idst, rows, zbuf, sem):
    c = lax.axis_index("c")
    sid = lax.axis_index("s")
    _init_zbuf(zbuf)
    srcs, dsts = [s0, s1, s2], [d0, d1, d2]
    outs = {1: o1, 2: o2, 4: o4, 3: o3, 0: o0, 5: o5}
    for jh in range(4):
        hc = c * 4 + jh
        for p in range(3):
            _zero_own(acc, zbuf, sid)
            plsc.subcore_barrier()
            G = GS[p]
            rps = G * 8
            sp, dp = srcs[p], dsts[p]

            def grp(g, carry):
                gb = sid * rps + g * 8
                pltpu.sync_copy(sp.at[hc, pl.ds(gb, 8)], isrc)
                pltpu.sync_copy(dp.at[pl.ds(gb, 8)], idst)
                hs = [pltpu.async_copy(tab8.at[isrc.at[j]], rows.at[j], sem)
                      for j in range(8)]
                for h in hs:
                    h.wait()
                for j in range(8):
                    pltpu.sync_copy(rows.at[j], acc.at[idst.at[j]], add=True)
                return carry

            lax.fori_loop(0, G, grp, 0)
            plsc.subcore_barrier()
            for (e, loff, npd) in PASSES[p]["members"]:
                sh = npd // 16
                pltpu.sync_copy(
                    acc.at[pl.ds(loff + sid * sh, sh)],
                    outs[e].at[pl.ds(sid * sh, sh), pl.ds(hc * 16, 16)])
            plsc.subcore_barrier()


_sc_l0 = pl.kernel(
    _sc_l0_body,
    out_type=tuple(jax.ShapeDtypeStruct((2, NPAD_T[DST_T[e]], 16), jnp.float32)
                   for e in EOUT),
    mesh=_MESH,
    scratch_types=[
        pltpu.VMEM_SHARED((SCRATCH_ROWS, 16), jnp.float32),
        pltpu.VMEM((4, 128), jnp.int32),
        pltpu.VMEM((4, 128), jnp.int32),
        pltpu.VMEM((4, 128, 16), jnp.float32),
        pltpu.VMEM((ZROWS, 16), jnp.float32),
        pltpu.SemaphoreType.DMA,
    ],
    compiler_params=_SC_PARAMS,
)

_sc_l1 = pl.kernel(
    _sc_l1_body,
    out_type=tuple(jax.ShapeDtypeStruct((NPAD_T[DST_T[e]], H), jnp.float32)
                   for e in EOUT),
    mesh=_MESH,
    scratch_types=[
        pltpu.VMEM_SHARED((SCRATCH_ROWS, 16), jnp.float32),
        pltpu.VMEM((8, 128), jnp.int32),
        pltpu.VMEM((8, 128), jnp.int32),
        pltpu.VMEM((8, 128, 16), jnp.float32),
        pltpu.VMEM((ZROWS, 16), jnp.float32),
        pltpu.SemaphoreType.DMA,
    ],
    compiler_params=_SC_PARAMS,
)


def _ln_relu(h, g, b):
    mu = jnp.mean(h, axis=-1, keepdims=True)
    var = jnp.mean((h - mu) ** 2, axis=-1, keepdims=True)
    h = (h - mu) * lax.rsqrt(var + 1e-5) * g + b
    return jnp.maximum(h, 0.0)


def _tc0_body(p0a, p1a, p0b, p1b, raw, A1, A2, Wself, bias, g, b, o):
    sa = p0a[...] + p1a[...]
    ma = sa / jnp.maximum(sa[:, 15:16], 1.0)
    sb = p0b[...] + p1b[...]
    mb = sb / jnp.maximum(sb[:, 15:16], 1.0)
    h = (jnp.dot(ma, A1[...], preferred_element_type=jnp.float32)
         + jnp.dot(mb, A2[...], preferred_element_type=jnp.float32)
         + jnp.dot(raw[...], Wself[...], preferred_element_type=jnp.float32)
         + bias[...])
    o[...] = _ln_relu(h, g[...], b[...])


def _tc1_body(sa, sb, q0a, q1a, q0b, q1b, h0, B1, B2, Wr1, bias, g, b, o):
    cnta = q0a[:, 15:16] + q1a[:, 15:16]
    cntb = q0b[:, 15:16] + q1b[:, 15:16]
    agga = sa[...] / jnp.maximum(cnta, 1.0)
    aggb = sb[...] / jnp.maximum(cntb, 1.0)
    h = (jnp.dot(agga, B1[...], preferred_element_type=jnp.float32)
         + jnp.dot(aggb, B2[...], preferred_element_type=jnp.float32)
         + jnp.dot(h0[...], Wr1[...], preferred_element_type=jnp.float32)
         + bias[...])
    o[...] = _ln_relu(h, g[...], b[...])


_BLK = 1024


def _rows_spec(width):
    return pl.BlockSpec((_BLK, width), lambda i: (i, 0))


def _full_spec(shape):
    return pl.BlockSpec(shape, lambda i: tuple(0 for _ in shape))


def _make_tc0(n_out):
    grid = (math.ceil(n_out / _BLK),)
    return pl.pallas_call(
        _tc0_body,
        grid=grid,
        in_specs=[_rows_spec(16)] * 5 + [
            _full_spec((16, H)), _full_spec((16, H)), _full_spec((16, H)),
            _full_spec((1, H)), _full_spec((1, H)), _full_spec((1, H))],
        out_specs=_rows_spec(H),
        out_shape=jax.ShapeDtypeStruct((n_out, H), jnp.float32),
    )


def _make_tc1(n_out):
    grid = (math.ceil(n_out / _BLK),)
    return pl.pallas_call(
        _tc1_body,
        grid=grid,
        in_specs=[_rows_spec(H), _rows_spec(H)] + [_rows_spec(16)] * 4 +
                 [_rows_spec(H),
                  _full_spec((H, H)), _full_spec((H, H)), _full_spec((H, H)),
                  _full_spec((1, H)), _full_spec((1, H)), _full_spec((1, H))],
        out_specs=_rows_spec(H),
        out_shape=jax.ShapeDtypeStruct((n_out, H), jnp.float32),
    )


_TC0 = [_make_tc0(n) for n in N_T]
_TC1 = [_make_tc1(n) for n in N_T]


def _pad16(x, npad):
    z = jnp.zeros((npad, 16), jnp.float32)
    z = z.at[:x.shape[0], :x.shape[1]].set(x)
    return z.at[:x.shape[0], 15].set(1.0)


def kernel(x_customer, x_product, x_store, Wc, bc, Wp, bp, Ws, bs, Wl, bl, Wr,
           ln_g, ln_b, edge_index_buys, edge_index_bought_by, edge_index_visits,
           edge_index_visited_by, edge_index_sold_at, edge_index_sells):
    edges = [edge_index_buys, edge_index_bought_by, edge_index_visits,
             edge_index_visited_by, edge_index_sold_at, edge_index_sells]
    raws = [x_customer, x_product, x_store]

    l0s, l1s, dsts = [], [], []
    for p in PASSES:
        sg = jnp.concatenate(
            [edges[e][0] + OFF[SRC_T[e]] for (e, _, _) in p["members"]])
        dl = jnp.concatenate(
            [edges[e][1] + loff for (e, loff, _) in p["members"]])
        padn = p["ep"] - sg.shape[0]
        sg = jnp.concatenate([sg, jnp.zeros((padn,), jnp.int32)])
        dl = jnp.concatenate([dl, jnp.full((padn,), DUMMY_ROW, jnp.int32)])
        l0s.append(sg.reshape(-1, 128))
        s8 = (sg * 8)[None, :] + jnp.arange(8, dtype=jnp.int32)[:, None]
        l1s.append(s8.reshape(8, -1, 128))
        dsts.append(dl.reshape(-1, 128))

    def wsrc_pad(t):
        W = [Wc, Wp, Ws][t]
        b = [bc, bp, bs][t]
        z = jnp.zeros((16, H), jnp.float32)
        z = z.at[:W.shape[0]].set(W)
        return z.at[15].set(b)

    WSP = [wsrc_pad(t) for t in range(3)]

    x16 = jnp.concatenate([_pad16(raws[t], N_T[t]) for t in range(3)], axis=0)
    raw16 = [_pad16(raws[t], NPAD_T[t]) for t in range(3)]

    l0o = _sc_l0(x16, l0s[0], dsts[0], l0s[1], dsts[1], l0s[2], dsts[2])
    q = {e: l0o[i] for i, e in enumerate(EOUT)}

    h0 = []
    for t in range(3):
        e1, e2 = DES[t]
        A1 = 0.5 * (WSP[SRC_T[e1]] @ Wl[0, e1])
        A2 = 0.5 * (WSP[SRC_T[e2]] @ Wl[0, e2])
        Wself = WSP[t] @ (0.5 * (Wr[0, e1] + Wr[0, e2]))
        bias = (0.5 * (bl[0, e1] + bl[0, e2])).reshape(1, H)
        h0.append(_TC0[t](
            q[e1][0], q[e1][1], q[e2][0], q[e2][1], raw16[t],
            A1, A2, Wself, bias,
            ln_g[0, t].reshape(1, H), ln_b[0, t].reshape(1, H)))

    tab8 = jnp.concatenate(h0, axis=0).reshape(NT_ALL * 8, 16)
    l1o = _sc_l1(tab8, l1s[0], dsts[0], l1s[1], dsts[1], l1s[2], dsts[2])
    sgm = {e: l1o[i] for i, e in enumerate(EOUT)}

    out = []
    for t in range(3):
        e1, e2 = DES[t]
        B1 = 0.5 * Wl[1, e1]
        B2 = 0.5 * Wl[1, e2]
        Wr1 = 0.5 * (Wr[1, e1] + Wr[1, e2])
        bias = (0.5 * (bl[1, e1] + bl[1, e2])).reshape(1, H)
        out.append(_TC1[t](
            sgm[e1], sgm[e2], q[e1][0], q[e1][1], q[e2][0], q[e2][1], h0[t],
            B1, B2, Wr1, bias,
            ln_g[1, t].reshape(1, H), ln_b[1, t].reshape(1, H)))
    return tuple(out)
